# Initial kernel scaffold; baseline (speedup 1.0000x reference)
import jax
import jax.numpy as jnp
from jax.experimental import pallas as pl


def kernel(features, edge_index, W, b):
    raise NotImplementedError("write your pallas kernel here")


# speedup vs baseline: 5.6335x; 5.6335x over previous
import dataclasses
import functools

import jax
import jax.numpy as jnp
from jax import lax
from jax.experimental import pallas as pl
from jax.experimental.pallas import tpu as pltpu
from jax.experimental.pallas import tpu_sc as plsc

NC = 2
NS = 16
NW = NC * NS
L = 16
CHUNK = 128
NBUF = 4


def _mesh():
    return plsc.VectorSubcoreMesh(core_axis_name="c", subcore_axis_name="s")


def _sc_params():
    cp = pltpu.CompilerParams(use_tc_tiling_on_sc=False)
    if "needs_layout_passes" in pltpu.CompilerParams.__dataclass_fields__:
        cp = dataclasses.replace(cp, needs_layout_passes=False)
    return cp


def _hist_call(Np, EPT):
    @functools.partial(
        pl.kernel,
        out_type=jax.ShapeDtypeStruct((NW, 2, Np), jnp.float32),
        mesh=_mesh(),
        compiler_params=_sc_params(),
        scratch_types=[
            pltpu.VMEM((Np,), jnp.float32),
            pltpu.VMEM((Np,), jnp.float32),
            pltpu.VMEM((EPT,), jnp.int32),
            pltpu.VMEM((EPT,), jnp.int32),
        ],
    )
    def hist(src_hbm, dst_hbm, zeros_hbm, out_hbm, hsrc, hdst, esrc, edst):
        c = lax.axis_index("c")
        s = lax.axis_index("s")
        wid = c * NS + s
        pltpu.sync_copy(zeros_hbm, hsrc)
        pltpu.sync_copy(zeros_hbm, hdst)
        pltpu.sync_copy(src_hbm.at[wid], esrc)
        pltpu.sync_copy(dst_hbm.at[wid], edst)
        ones = jnp.full((L,), 1.0, jnp.float32)

        @pl.loop(0, EPT, step=L)
        def _(j):
            plsc.addupdate_scatter(hsrc, [esrc[pl.ds(j, L)]], ones)
            plsc.addupdate_scatter(hdst, [edst[pl.ds(j, L)]], ones)

        pltpu.sync_copy(hsrc, out_hbm.at[wid, 0])
        pltpu.sync_copy(hdst, out_hbm.at[wid, 1])

    return hist


def _scale_call(Np, D):
    Dh = D // NC

    def body(x_ref, h_ref, o_ref):
        deg = jnp.sum(h_ref[:, 0, :], axis=0)
        s = lax.rsqrt(jnp.maximum(deg, 1.0))
        scaled = x_ref[...] * s[:, None]
        for c in range(NC):
            o_ref[c] = scaled[:, c * Dh:(c + 1) * Dh]

    return pl.pallas_call(
        body, out_shape=jax.ShapeDtypeStruct((NC, Np, Dh), jnp.float32)
    )


def _agg_call(Np, CPT, D):
    Dh = D // NC
    rows_per_tile = Np // NS

    @functools.partial(
        pl.kernel,
        out_type=jax.ShapeDtypeStruct((NC, Np, Dh), jnp.float32),
        mesh=_mesh(),
        compiler_params=_sc_params(),
        scratch_types=[
            pltpu.VMEM((CPT, CHUNK), jnp.int32),
            pltpu.VMEM((CPT, CHUNK), jnp.int32),
            [pltpu.VMEM((CHUNK, Dh), jnp.float32) for _ in range(NBUF)],
            pltpu.VMEM_SHARED((Np, Dh), jnp.float32),
            [pltpu.SemaphoreType.DMA for _ in range(NBUF)],
            [pltpu.SemaphoreType.DMA for _ in range(NBUF)],
        ],
    )
    def agg(feat_hbm, src_hbm, dst_hbm, zrows_hbm, out_hbm,
            sidx, didx, bufs, accum, gsems, ssems):
        c = lax.axis_index("c")
        s = lax.axis_index("s")
        my_rows = pl.ds(s * rows_per_tile, rows_per_tile)
        pltpu.sync_copy(zrows_hbm.at[my_rows], accum.at[my_rows])
        pltpu.sync_copy(src_hbm.at[s], sidx)
        pltpu.sync_copy(dst_hbm.at[s], didx)
        plsc.subcore_barrier()
        feat_c = feat_hbm.at[c]

        def wait_gather(b, ch):
            pltpu.make_async_copy(feat_c.at[sidx.at[ch]], bufs[b], gsems[b]).wait()

        def scatter_add(b, ch):
            pltpu.async_copy(bufs[b], accum.at[didx.at[ch]], ssems[b],
                             add=True).wait()

        for b in range(NBUF):
            pltpu.async_copy(feat_c.at[sidx.at[b]], bufs[b], gsems[b])

        @pl.loop(0, CPT - NBUF, step=NBUF)
        def _(c0):
            for b in range(NBUF):
                ch = c0 + b
                wait_gather(b, ch)
                scatter_add(b, ch)
                pltpu.async_copy(feat_c.at[sidx.at[ch + NBUF]], bufs[b],
                                 gsems[b])

        for b in range(NBUF):
            ch = CPT - NBUF + b
            wait_gather(b, ch)
            scatter_add(b, ch)

        plsc.subcore_barrier()
        pltpu.sync_copy(accum.at[my_rows], out_hbm.at[c].at[my_rows])

    return agg


def _out_call(Np, D):
    def body(a_ref, h_ref, w_ref, b_ref, o_ref):
        a = jnp.concatenate([a_ref[c] for c in range(NC)], axis=1)
        deg = jnp.sum(h_ref[:, 1, :], axis=0)
        s = lax.rsqrt(jnp.maximum(deg, 1.0))
        x = a * s[:, None]
        y = jnp.dot(x, w_ref[...], preferred_element_type=jnp.float32)
        o_ref[...] = jnp.maximum(y + b_ref[...], 0.0)

    return pl.pallas_call(
        body, out_shape=jax.ShapeDtypeStruct((Np, D), jnp.float32)
    )


def kernel(features, edge_index, W, b):
    N, D = features.shape
    E = edge_index.shape[1]
    Np = -(-(N + 1) // 2048) * 2048
    grain = CHUNK * NBUF
    EPT = -(-E // (NS * grain)) * grain
    CPT = EPT // CHUNK
    E_pad = NS * EPT

    pad = jnp.full((E_pad - E,), N, jnp.int32)
    srcp = jnp.concatenate([edge_index[0], pad])
    dstp = jnp.concatenate([edge_index[1], pad])
    featp = jnp.pad(features.astype(jnp.float32), ((0, Np - N), (0, 0)))
    zeros1 = jnp.zeros((Np,), jnp.float32)
    zeros2 = jnp.zeros((Np, D // NC), jnp.float32)

    hists = _hist_call(Np, E_pad // NW)(srcp.reshape(NW, E_pad // NW),
                                        dstp.reshape(NW, E_pad // NW), zeros1)
    feat_scaled = _scale_call(Np, D)(featp, hists)
    agg = _agg_call(Np, CPT, D)(feat_scaled, srcp.reshape(NS, CPT, CHUNK),
                                dstp.reshape(NS, CPT, CHUNK), zeros2)
    out = _out_call(Np, D)(agg, hists, W.astype(jnp.float32),
                           b.astype(jnp.float32).reshape(1, D))
    return out[:N]


# speedup vs baseline: 6.2676x; 1.1126x over previous
import dataclasses
import functools

import jax
import jax.numpy as jnp
from jax import lax
from jax.experimental import pallas as pl
from jax.experimental.pallas import tpu as pltpu
from jax.experimental.pallas import tpu_sc as plsc

NC = 2
NS = 16
NW = NC * NS
L = 16
CHUNK = 128
NBUF = 4


def _mesh():
    return plsc.VectorSubcoreMesh(core_axis_name="c", subcore_axis_name="s")


def _sc_params():
    cp = pltpu.CompilerParams(use_tc_tiling_on_sc=False)
    if "needs_layout_passes" in pltpu.CompilerParams.__dataclass_fields__:
        cp = dataclasses.replace(cp, needs_layout_passes=False)
    return cp


def _matmul_call(Np, D):
    def body(x_ref, w_ref, o_ref):
        o_ref[...] = jnp.dot(x_ref[...], w_ref[...],
                             preferred_element_type=jnp.float32)

    return pl.pallas_call(
        body, out_shape=jax.ShapeDtypeStruct((Np, D), jnp.float32)
    )


def _hist_call(Np, EPT):
    @functools.partial(
        pl.kernel,
        out_type=jax.ShapeDtypeStruct((NW, 2, Np), jnp.float32),
        mesh=_mesh(),
        compiler_params=_sc_params(),
        scratch_types=[
            pltpu.VMEM((Np,), jnp.float32),
            pltpu.VMEM((Np,), jnp.float32),
            pltpu.VMEM((EPT,), jnp.int32),
            pltpu.VMEM((EPT,), jnp.int32),
        ],
    )
    def hist(src_hbm, dst_hbm, zeros_hbm, out_hbm, hsrc, hdst, esrc, edst):
        c = lax.axis_index("c")
        s = lax.axis_index("s")
        wid = c * NS + s
        pltpu.sync_copy(zeros_hbm, hsrc)
        pltpu.sync_copy(zeros_hbm, hdst)
        pltpu.sync_copy(src_hbm.at[wid], esrc)
        pltpu.sync_copy(dst_hbm.at[wid], edst)
        ones = jnp.full((L,), 1.0, jnp.float32)

        @pl.loop(0, EPT, step=L)
        def _(j):
            plsc.addupdate_scatter(hsrc, [esrc[pl.ds(j, L)]], ones)
            plsc.addupdate_scatter(hdst, [edst[pl.ds(j, L)]], ones)

        pltpu.sync_copy(hsrc, out_hbm.at[wid, 0])
        pltpu.sync_copy(hdst, out_hbm.at[wid, 1])

    return hist


def _scale_call(Np, D):
    def body(y_ref, h_ref, o_ref):
        deg = jnp.sum(h_ref[:, 0, :], axis=0)
        s = lax.rsqrt(jnp.maximum(deg, 1.0))
        o_ref[...] = (y_ref[...] * s[:, None]).astype(jnp.bfloat16)

    return pl.pallas_call(
        body, out_shape=jax.ShapeDtypeStruct((Np, D), jnp.bfloat16)
    )


def _agg_call(Np, CPT, D):
    rows_per_tile = Np // NS

    @functools.partial(
        pl.kernel,
        out_type=jax.ShapeDtypeStruct((NC, Np, D), jnp.bfloat16),
        mesh=_mesh(),
        compiler_params=_sc_params(),
        scratch_types=[
            pltpu.VMEM((CPT, CHUNK), jnp.int32),
            pltpu.VMEM((CPT, CHUNK), jnp.int32),
            [pltpu.VMEM((CHUNK, D), jnp.bfloat16) for _ in range(NBUF)],
            pltpu.VMEM_SHARED((Np, D), jnp.bfloat16),
            [pltpu.SemaphoreType.DMA for _ in range(NBUF)],
            [pltpu.SemaphoreType.DMA for _ in range(NBUF)],
        ],
    )
    def agg(feat_hbm, src_hbm, dst_hbm, zrows_hbm, out_hbm,
            sidx, didx, bufs, accum, gsems, ssems):
        c = lax.axis_index("c")
        s = lax.axis_index("s")
        wid = c * NS + s
        my_rows = pl.ds(s * rows_per_tile, rows_per_tile)
        pltpu.sync_copy(zrows_hbm.at[my_rows], accum.at[my_rows])
        pltpu.sync_copy(src_hbm.at[wid], sidx)
        pltpu.sync_copy(dst_hbm.at[wid], didx)
        plsc.subcore_barrier()

        def wait_gather(b, ch):
            pltpu.make_async_copy(feat_hbm.at[sidx.at[ch]], bufs[b],
                                  gsems[b]).wait()

        def scatter_add(b, ch):
            pltpu.async_copy(bufs[b], accum.at[didx.at[ch]], ssems[b],
                             add=True).wait()

        for b in range(NBUF):
            pltpu.async_copy(feat_hbm.at[sidx.at[b]], bufs[b], gsems[b])

        @pl.loop(0, CPT - NBUF, step=NBUF)
        def _(c0):
            for b in range(NBUF):
                ch = c0 + b
                wait_gather(b, ch)
                scatter_add(b, ch)
                pltpu.async_copy(feat_hbm.at[sidx.at[ch + NBUF]], bufs[b],
                                 gsems[b])

        for b in range(NBUF):
            ch = CPT - NBUF + b
            wait_gather(b, ch)
            scatter_add(b, ch)

        plsc.subcore_barrier()
        pltpu.sync_copy(accum.at[my_rows], out_hbm.at[c].at[my_rows])

    return agg


def _out_call(Np, D):
    def body(a_ref, h_ref, b_ref, o_ref):
        a = a_ref[0].astype(jnp.float32) + a_ref[1].astype(jnp.float32)
        deg = jnp.sum(h_ref[:, 1, :], axis=0)
        s = lax.rsqrt(jnp.maximum(deg, 1.0))
        o_ref[...] = jnp.maximum(a * s[:, None] + b_ref[...], 0.0)

    return pl.pallas_call(
        body, out_shape=jax.ShapeDtypeStruct((Np, D), jnp.float32)
    )


def kernel(features, edge_index, W, b):
    N, D = features.shape
    E = edge_index.shape[1]
    Np = -(-(N + 1) // 2048) * 2048
    grain = CHUNK * NBUF
    EPT = -(-E // (NW * grain)) * grain
    CPT = EPT // CHUNK
    E_pad = NW * EPT

    pad = jnp.full((E_pad - E,), N, jnp.int32)
    srcp = jnp.concatenate([edge_index[0], pad])
    dstp = jnp.concatenate([edge_index[1], pad])
    featp = jnp.pad(features.astype(jnp.float32), ((0, Np - N), (0, 0)))
    zeros1 = jnp.zeros((Np,), jnp.float32)
    zeros2 = jnp.zeros((Np, D), jnp.bfloat16)

    y = _matmul_call(Np, D)(featp, W.astype(jnp.float32))
    hists = _hist_call(Np, EPT)(srcp.reshape(NW, EPT), dstp.reshape(NW, EPT),
                                zeros1)
    yb = _scale_call(Np, D)(y, hists)
    agg = _agg_call(Np, CPT, D)(yb, srcp.reshape(NW, CPT, CHUNK),
                                dstp.reshape(NW, CPT, CHUNK), zeros2)
    out = _out_call(Np, D)(agg, hists, b.astype(jnp.float32).reshape(1, D))
    return out[:N]
